# baseline (speedup 1.0000x reference)
import functools

import jax
import jax.numpy as jnp
from jax import lax
from jax.experimental import pallas as pl
from jax.experimental.pallas import tpu as pltpu
from jax.experimental.pallas import tpu_sc as plsc

N = 10000
E = 320000
D = 128

NC = 2
NS = 16
NW = NC * NS
EPW = E // NW
K = 40
NV = 10
NB = EPW // (K * NV)
NPAD = 10240
RPT = NPAD // NS


def _leaky(v):
    return jnp.maximum(v, 0.01 * v)


def _prep_body(x_ref, pos_ref, w1t_ref, b1_ref, w2t_ref, b2_ref,
               wfrt_ref, wfxt_ref, bf_ref, t_ref):
    x = x_ref[...]
    pos = pos_ref[...]
    h = _leaky(jnp.dot(x, w1t_ref[...], preferred_element_type=jnp.float32)
               + b1_ref[...])
    delta = jnp.tanh(
        jnp.dot(h, w2t_ref[...], preferred_element_type=jnp.float32)
        + b2_ref[...])
    wfrt = wfrt_ref[...]
    t_ref[pl.ds(0, N), :] = (
        jnp.dot(x, wfxt_ref[...], preferred_element_type=jnp.float32)
        + jnp.dot(pos, wfrt, preferred_element_type=jnp.float32))
    t_ref[pl.ds(N, N), :] = (
        jnp.dot(delta - pos, wfrt, preferred_element_type=jnp.float32)
        + bf_ref[...])


def _post_body(p_ref, wg1t_ref, bg1_ref, wg2t_ref, bg2_ref, out_ref):
    agg = p_ref[0, :N] + p_ref[1, :N]
    t = _leaky(jnp.dot(agg, wg1t_ref[...], preferred_element_type=jnp.float32)
               + bg1_ref[...])
    out_ref[...] = (jnp.dot(t, wg2t_ref[...],
                            preferred_element_type=jnp.float32)
                    + bg2_ref[...])


_mesh = plsc.VectorSubcoreMesh(core_axis_name="c", subcore_axis_name="s")


@functools.partial(
    pl.kernel,
    mesh=_mesh,
    out_type=jax.ShapeDtypeStruct((NC, NPAD, D), jnp.float32),
    scratch_types=[
        pltpu.VMEM((NB, 1, 2 * K), jnp.int32),
        pltpu.VMEM((NB, 1, K), jnp.int32),
        pltpu.VMEM((2 * K, D), jnp.float32),
        pltpu.VMEM((2 * K, D), jnp.float32),
        pltpu.VMEM((2 * K, D), jnp.float32),
        pltpu.VMEM((K, D), jnp.float32),
        pltpu.VMEM((K, D), jnp.float32),
        pltpu.VMEM_SHARED((NPAD, D), jnp.float32),
        pltpu.SemaphoreType.DMA((3,)),
        pltpu.SemaphoreType.DMA((2,)),
    ],
)
def _edge_kernel(t_hbm, gidx_hbm, dst_hbm, zero_hbm, out_hbm,
                 gidx_v, dst_v, ab0, ab1, ab2, e0, e1,
                 agg_sh, sga, ses):
    c = lax.axis_index("c")
    s = lax.axis_index("s")
    wid = s * NC + c

    pltpu.sync_copy(zero_hbm, agg_sh.at[pl.ds(s * RPT, RPT)])
    plsc.subcore_barrier()

    gab = (ab0, ab1, ab2)
    ge = (e0, e1)

    def issue_gathers(t, gs):
        pltpu.async_copy(t_hbm.at[gidx_v.at[t, 0]], gab[gs], sga.at[gs])

    def wait_gathers(gs):
        pltpu.make_async_copy(t_hbm.at[gidx_v.at[0, 0]], gab[gs],
                              sga.at[gs]).wait()

    def issue_scatter(t, es):
        pltpu.async_copy(ge[es], agg_sh.at[dst_v.at[t, 0]], ses.at[es],
                         add=True)

    def wait_scatter(es):
        pltpu.make_async_copy(ge[es], agg_sh.at[dst_v.at[0, 0]],
                              ses.at[es]).wait()

    def compute_block(gs, es):
        ab_r, e_r = gab[gs], ge[es]

        def row(r, rc):
            for j in range(D // 16):
                sl = pl.ds(j * 16, 16)
                m = ab_r[r, sl] + ab_r[K + r, sl]
                e_r[r, sl] = jnp.maximum(m, m * 0.01)
            return rc

        lax.fori_loop(0, K, row, 0)

    for v in range(NV):
        cs = pltpu.async_copy(gidx_hbm.at[wid, v], gidx_v, sga.at[0])
        cd = pltpu.async_copy(dst_hbm.at[wid, v], dst_v, sga.at[1])
        cs.wait()
        cd.wait()

        issue_gathers(0, 0)
        issue_gathers(1, 1)
        issue_gathers(2, 2)

        def outer(t0, carry):
            for i in range(6):
                t = 6 * t0 + i
                gs = i % 3
                es = i % 2
                wait_gathers(gs)

                @pl.when(t >= 2)
                def _():
                    wait_scatter(es)

                compute_block(gs, es)
                issue_scatter(t, es)
                if i < 4:
                    issue_gathers(t + 3, gs)
                else:
                    @pl.when(t0 < NB // 6 - 1)
                    def _():
                        issue_gathers(t + 3, gs)
            return carry

        lax.fori_loop(0, NB // 6, outer, 0)

        wait_gathers(0)
        wait_scatter(0)
        compute_block(0, 0)
        issue_scatter(NB - 1, 0)

        wait_scatter(1)
        wait_scatter(0)

    plsc.subcore_barrier()
    pltpu.sync_copy(agg_sh.at[pl.ds(s * RPT, RPT)],
                    out_hbm.at[c, pl.ds(s * RPT, RPT)])


def kernel(x, pos, edge_index, W1h, b1h, W2h, b2h, Wf, bf, Wg1, bg1, Wg2, bg2):
    x = x.astype(jnp.float32)
    pos = pos.astype(jnp.float32)
    src = edge_index[0].astype(jnp.int32).reshape(NW, NV, NB, 1, K)
    dst = edge_index[1].astype(jnp.int32).reshape(NW, NV, NB, 1, K)
    gidx = jnp.concatenate([src, dst + N], axis=4)

    tbl = pl.pallas_call(
        _prep_body,
        out_shape=jax.ShapeDtypeStruct((2 * N, D), jnp.float32),
    )(x, pos, W1h.T, b1h.reshape(1, D), W2h.T, b2h.reshape(1, 3),
      Wf[:, :3].T, Wf[:, 3:].T, bf.reshape(1, D))

    zeros = jnp.zeros((RPT, D), jnp.float32)
    partials = _edge_kernel(tbl, gidx, dst, zeros)

    out = pl.pallas_call(
        _post_body,
        out_shape=jax.ShapeDtypeStruct((N, D), jnp.float32),
    )(partials, Wg1.T, bg1.reshape(1, D), Wg2.T, bg2.reshape(1, D))
    return out


# speedup vs baseline: 1.0803x; 1.0803x over previous
import functools

import jax
import jax.numpy as jnp
from jax import lax
from jax.experimental import pallas as pl
from jax.experimental.pallas import tpu as pltpu
from jax.experimental.pallas import tpu_sc as plsc

N = 10000
E = 320000
D = 128

NC = 2
NS = 16
NW = NC * NS
EPW = E // NW
K = 40
NV = 10
NB = EPW // (K * NV)
NPAD = 10240
RPT = NPAD // NS


def _leaky(v):
    return jnp.maximum(v, 0.01 * v)


def _prep_body(x_ref, pos_ref, w1t_ref, b1_ref, w2t_ref, b2_ref,
               wfrt_ref, wfxt_ref, bf_ref, a_ref, b_ref):
    x = x_ref[...]
    pos = pos_ref[...]
    h = _leaky(jnp.dot(x, w1t_ref[...], preferred_element_type=jnp.float32)
               + b1_ref[...])
    delta = jnp.tanh(
        jnp.dot(h, w2t_ref[...], preferred_element_type=jnp.float32)
        + b2_ref[...])
    wfrt = wfrt_ref[...]
    a_ref[...] = (jnp.dot(x, wfxt_ref[...], preferred_element_type=jnp.float32)
                  + jnp.dot(pos, wfrt, preferred_element_type=jnp.float32))
    b_ref[...] = (jnp.dot(delta - pos, wfrt,
                          preferred_element_type=jnp.float32)
                  + bf_ref[...])


def _post_body(p_ref, wg1t_ref, bg1_ref, wg2t_ref, bg2_ref, out_ref):
    agg = p_ref[0, :N] + p_ref[1, :N]
    t = _leaky(jnp.dot(agg, wg1t_ref[...], preferred_element_type=jnp.float32)
               + bg1_ref[...])
    out_ref[...] = (jnp.dot(t, wg2t_ref[...],
                            preferred_element_type=jnp.float32)
                    + bg2_ref[...])


_mesh = plsc.VectorSubcoreMesh(core_axis_name="c", subcore_axis_name="s")


@functools.partial(
    pl.kernel,
    mesh=_mesh,
    out_type=jax.ShapeDtypeStruct((NC, NPAD, D), jnp.float32),
    scratch_types=[
        pltpu.VMEM((NB, 1, K), jnp.int32),
        pltpu.VMEM((NB, 1, K), jnp.int32),
        pltpu.VMEM((K, D), jnp.float32),
        pltpu.VMEM((K, D), jnp.float32),
        pltpu.VMEM((K, D), jnp.float32),
        pltpu.VMEM((K, D), jnp.float32),
        pltpu.VMEM((K, D), jnp.float32),
        pltpu.VMEM((K, D), jnp.float32),
        pltpu.VMEM((K, D), jnp.float32),
        pltpu.VMEM((K, D), jnp.float32),
        pltpu.VMEM_SHARED((NPAD, D), jnp.float32),
        pltpu.SemaphoreType.DMA((3,)),
        pltpu.SemaphoreType.DMA((3,)),
        pltpu.SemaphoreType.DMA((2,)),
    ],
)
def _edge_kernel(a_hbm, b_hbm, src_hbm, dst_hbm, zero_hbm, out_hbm,
                 src_v, dst_v, a0, a1, a2, b0, b1, b2, e0, e1,
                 agg_sh, sga, sgb, ses):
    c = lax.axis_index("c")
    s = lax.axis_index("s")
    wid = s * NC + c

    pltpu.sync_copy(zero_hbm, agg_sh.at[pl.ds(s * RPT, RPT)])
    plsc.subcore_barrier()

    ga = (a0, a1, a2)
    gb = (b0, b1, b2)
    ge = (e0, e1)

    def issue_gathers(t, gs):
        pltpu.async_copy(a_hbm.at[src_v.at[t, 0]], ga[gs], sga.at[gs])
        pltpu.async_copy(b_hbm.at[dst_v.at[t, 0]], gb[gs], sgb.at[gs])

    def wait_gathers(gs):
        pltpu.make_async_copy(a_hbm.at[src_v.at[0, 0]], ga[gs],
                              sga.at[gs]).wait()
        pltpu.make_async_copy(b_hbm.at[dst_v.at[0, 0]], gb[gs],
                              sgb.at[gs]).wait()

    def issue_scatter(t, es):
        pltpu.async_copy(ge[es], agg_sh.at[dst_v.at[t, 0]], ses.at[es],
                         add=True)

    def wait_scatter(es):
        pltpu.make_async_copy(ge[es], agg_sh.at[dst_v.at[0, 0]],
                              ses.at[es]).wait()

    def compute_block(gs, es):
        a_r, b_r, e_r = ga[gs], gb[gs], ge[es]

        def row(r, rc):
            for j in range(D // 16):
                sl = pl.ds(j * 16, 16)
                m = a_r[r, sl] + b_r[r, sl]
                e_r[r, sl] = jnp.maximum(m, m * 0.01)
            return rc

        lax.fori_loop(0, K, row, 0)

    for v in range(NV):
        cs = pltpu.async_copy(src_hbm.at[wid, v], src_v, sga.at[0])
        cd = pltpu.async_copy(dst_hbm.at[wid, v], dst_v, sgb.at[0])
        cs.wait()
        cd.wait()

        issue_gathers(0, 0)
        issue_gathers(1, 1)
        issue_gathers(2, 2)

        def outer(t0, carry):
            for i in range(6):
                t = 6 * t0 + i
                gs = i % 3
                es = i % 2
                wait_gathers(gs)

                @pl.when(t >= 2)
                def _():
                    wait_scatter(es)

                compute_block(gs, es)
                issue_scatter(t, es)
                if i < 4:
                    issue_gathers(t + 3, gs)
                else:
                    @pl.when(t0 < NB // 6 - 1)
                    def _():
                        issue_gathers(t + 3, gs)
            return carry

        lax.fori_loop(0, NB // 6, outer, 0)

        wait_gathers(0)
        wait_scatter(0)
        compute_block(0, 0)
        issue_scatter(NB - 1, 0)

        wait_scatter(1)
        wait_scatter(0)

    plsc.subcore_barrier()
    pltpu.sync_copy(agg_sh.at[pl.ds(s * RPT, RPT)],
                    out_hbm.at[c, pl.ds(s * RPT, RPT)])


def kernel(x, pos, edge_index, W1h, b1h, W2h, b2h, Wf, bf, Wg1, bg1, Wg2, bg2):
    x = x.astype(jnp.float32)
    pos = pos.astype(jnp.float32)
    src = edge_index[0].astype(jnp.int32).reshape(NW, NV, NB, 1, K)
    dst = edge_index[1].astype(jnp.int32).reshape(NW, NV, NB, 1, K)

    a, b = pl.pallas_call(
        _prep_body,
        out_shape=[jax.ShapeDtypeStruct((N, D), jnp.float32),
                   jax.ShapeDtypeStruct((N, D), jnp.float32)],
    )(x, pos, W1h.T, b1h.reshape(1, D), W2h.T, b2h.reshape(1, 3),
      Wf[:, :3].T, Wf[:, 3:].T, bf.reshape(1, D))

    zeros = jnp.zeros((RPT, D), jnp.float32)
    partials = _edge_kernel(a, b, src, dst, zeros)

    out = pl.pallas_call(
        _post_body,
        out_shape=jax.ShapeDtypeStruct((N, D), jnp.float32),
    )(partials, Wg1.T, bg1.reshape(1, D), Wg2.T, bg2.reshape(1, D))
    return out
